# baseline (speedup 1.0000x reference)
import jax
import jax.numpy as jnp
from jax import lax
from jax.experimental import pallas as pl
from jax.experimental.pallas import tpu as pltpu
from jax.experimental.pallas import tpu_sc as plsc

GRID_NUNQ = 384
HIDDEN = 128
HIST = 200
TB = 128

_SC_INFO = plsc.get_sparse_core_info()
_NC = _SC_INFO.num_cores
_NS = _SC_INFO.num_subcores
_NW = _NC * _NS


def _sc_gather(table_hbm, idx_hbm, out_hbm, idx_v, rows_v, sem):
    b_per_w = idx_v.shape[0]
    wid = lax.axis_index("s") * _NC + lax.axis_index("c")
    base = wid * b_per_w
    pltpu.sync_copy(idx_hbm.at[pl.ds(base, b_per_w)], idx_v)
    pltpu.async_copy(table_hbm.at[idx_v], rows_v, sem).wait()
    pltpu.sync_copy(rows_v, out_hbm.at[pl.ds(base, b_per_w)])


def _gather_rows(embedding, g_id):
    batch = g_id.shape[0]
    b_per_w = batch // _NW
    mesh = plsc.VectorSubcoreMesh(core_axis_name="c", subcore_axis_name="s")
    return pl.kernel(
        _sc_gather,
        mesh=mesh,
        out_type=jax.ShapeDtypeStruct((batch, HIDDEN), jnp.float32),
        scratch_types=[
            pltpu.VMEM((b_per_w,), jnp.int32),
            pltpu.VMEM((b_per_w, HIDDEN), jnp.float32),
            pltpu.SemaphoreType.DMA,
        ],
    )(embedding, g_id)


def _tc_body(gid_ref, x_ref, emb_ref, eg_ref, o_ref):
    j = pl.program_id(0)

    @pl.when(j == 0)
    def _onehot_half():
        ids = gid_ref[0]
        one_hot = (
            jax.lax.broadcasted_iota(jnp.int32, (GRID_NUNQ, TB), 0) == ids
        ).astype(jnp.float32)
        emb_tile = jax.lax.dot_general(
            one_hot,
            emb_ref[...],
            (((0,), (0,)), ((), ())),
            preferred_element_type=jnp.float32,
        )
        o_ref[...] = x_ref[...] + emb_tile[:, None, :]

    @pl.when(j == 1)
    def _sc_half():
        o_ref[...] = x_ref[...] + eg_ref[...][:, None, :]


@jax.jit
def kernel(x, g_id, embedding):
    batch = x.shape[0]
    half = batch // 2
    nb = half // TB
    gid32 = g_id.astype(jnp.int32)
    gid3 = gid32[:half].reshape(nb, 1, TB)

    emb_g1 = _gather_rows(embedding, gid32[half:])

    return pl.pallas_call(
        _tc_body,
        grid=(2, nb),
        in_specs=[
            pl.BlockSpec((1, 1, TB), lambda j, i: (i, 0, 0)),
            pl.BlockSpec((TB, HIST, HIDDEN), lambda j, i: (j * nb + i, 0, 0)),
            pl.BlockSpec((GRID_NUNQ, HIDDEN), lambda j, i: (0, 0)),
            pl.BlockSpec((TB, HIDDEN), lambda j, i: (i, 0)),
        ],
        out_specs=pl.BlockSpec(
            (TB, HIST, HIDDEN), lambda j, i: (j * nb + i, 0, 0)
        ),
        out_shape=jax.ShapeDtypeStruct((batch, HIST, HIDDEN), jnp.float32),
        compiler_params=pltpu.CompilerParams(
            dimension_semantics=("arbitrary", "arbitrary"),
        ),
    )(gid3, x, embedding, emb_g1)


# speedup vs baseline: 1.0056x; 1.0056x over previous
import jax
import jax.numpy as jnp
from jax import lax
from jax.experimental import pallas as pl
from jax.experimental.pallas import tpu as pltpu
from jax.experimental.pallas import tpu_sc as plsc

GRID_NUNQ = 384
HIDDEN = 128
HIST = 200
TB = 128

_SC_INFO = plsc.get_sparse_core_info()
_NC = _SC_INFO.num_cores
_NS = _SC_INFO.num_subcores
_NW = _NC * _NS


def _sc_gather(table_hbm, idx_hbm, out_hbm, idx_v, rows_v, sem):
    b_per_w = idx_v.shape[0]
    wid = lax.axis_index("s") * _NC + lax.axis_index("c")
    base = wid * b_per_w
    pltpu.sync_copy(idx_hbm.at[pl.ds(base, b_per_w)], idx_v)
    pltpu.async_copy(table_hbm.at[idx_v], rows_v, sem).wait()
    pltpu.sync_copy(rows_v, out_hbm.at[pl.ds(base, b_per_w)])


def _gather_rows(embedding, g_id):
    batch = g_id.shape[0]
    b_per_w = batch // _NW
    mesh = plsc.VectorSubcoreMesh(core_axis_name="c", subcore_axis_name="s")
    return pl.kernel(
        _sc_gather,
        mesh=mesh,
        out_type=jax.ShapeDtypeStruct((batch, HIDDEN), jnp.float32),
        scratch_types=[
            pltpu.VMEM((b_per_w,), jnp.int32),
            pltpu.VMEM((b_per_w, HIDDEN), jnp.float32),
            pltpu.SemaphoreType.DMA,
        ],
    )(embedding, g_id)


def _tc_onehot_add(gid_ref, x_ref, emb_ref, o_ref):
    ids = gid_ref[0]
    one_hot = (
        jax.lax.broadcasted_iota(jnp.int32, (GRID_NUNQ, TB), 0) == ids
    ).astype(jnp.float32)
    emb_tile = jax.lax.dot_general(
        one_hot,
        emb_ref[...],
        (((0,), (0,)), ((), ())),
        preferred_element_type=jnp.float32,
    )
    o_ref[...] = x_ref[...] + emb_tile[:, None, :]


def _tc_add2(x_ref, eg_ref, buf_ref, o_ref):
    del buf_ref
    o_ref[...] = x_ref[...] + eg_ref[...][:, None, :]


@jax.jit
def kernel(x, g_id, embedding):
    batch = x.shape[0]
    half = batch // 2
    nb = half // TB
    gid32 = g_id.astype(jnp.int32)
    gid3 = gid32[:half].reshape(nb, 1, TB)

    emb_g1 = _gather_rows(embedding, gid32[half:])

    buf = pl.pallas_call(
        _tc_onehot_add,
        grid=(nb,),
        in_specs=[
            pl.BlockSpec((1, 1, TB), lambda i: (i, 0, 0)),
            pl.BlockSpec((TB, HIST, HIDDEN), lambda i: (i, 0, 0)),
            pl.BlockSpec((GRID_NUNQ, HIDDEN), lambda i: (0, 0)),
        ],
        out_specs=pl.BlockSpec((TB, HIST, HIDDEN), lambda i: (i, 0, 0)),
        out_shape=jax.ShapeDtypeStruct((batch, HIST, HIDDEN), jnp.float32),
        compiler_params=pltpu.CompilerParams(
            dimension_semantics=("arbitrary",),
        ),
    )(gid3, x, embedding)

    out = pl.pallas_call(
        _tc_add2,
        grid=(nb,),
        in_specs=[
            pl.BlockSpec((TB, HIST, HIDDEN), lambda i: (i + nb, 0, 0)),
            pl.BlockSpec((TB, HIDDEN), lambda i: (i, 0)),
            pl.BlockSpec(memory_space=pl.ANY),
        ],
        out_specs=pl.BlockSpec((TB, HIST, HIDDEN), lambda i: (i + nb, 0, 0)),
        out_shape=jax.ShapeDtypeStruct((batch, HIST, HIDDEN), jnp.float32),
        input_output_aliases={2: 0},
        compiler_params=pltpu.CompilerParams(
            dimension_semantics=("arbitrary",),
        ),
    )(x, emb_g1, buf)
    return out
